# baseline (speedup 1.0000x reference)
import functools
import math

import jax
import jax.numpy as jnp
import numpy as np
from jax import lax
from jax.experimental import pallas as pl
from jax.experimental.pallas import tpu as pltpu
from jax.experimental.pallas import tpu_sc as plsc

_B = 128
_V = 100000
_L = 8
_SMOOTHING = 0.1
_BASE = float(np.float32(_SMOOTHING / (_V - _L)))
_HI = float(np.float32((1.0 - _SMOOTHING) / _L))
_C_UNIQ = _HI * math.log(_HI) - _BASE * math.log(_BASE)
_C_ENT = _B * _V * _BASE * math.log(_BASE)

_NC, _NS = 2, 16
_NW = _NC * _NS
_RPW = _B // _NW
_IPW = _RPW * _L

_RBLK = 25000
_NBLK = _V // _RBLK


def _sc_label_partials(labels_flat, out_flat):
    mesh = plsc.VectorSubcoreMesh(core_axis_name="c", subcore_axis_name="s")

    @functools.partial(
        pl.kernel,
        mesh=mesh,
        out_type=(
            jax.ShapeDtypeStruct((_NW, 16), jnp.float32),
            jax.ShapeDtypeStruct((_NW, 16), jnp.float32),
        ),
        scratch_types=[
            pltpu.VMEM((_IPW,), jnp.int32),
            pltpu.VMEM((_IPW,), jnp.int32),
            pltpu.VMEM((_IPW,), jnp.float32),
            pltpu.VMEM((16,), jnp.float32),
            pltpu.VMEM((16,), jnp.float32),
            pltpu.SemaphoreType.DMA,
        ],
    )
    def sc_kernel(labels_hbm, x_hbm, g_hbm, u_hbm,
                  lab_v, idx_v, val_v, gst_v, ust_v, sem):
        wid = lax.axis_index("s") * _NC + lax.axis_index("c")
        base = wid * _IPW
        pltpu.sync_copy(labels_hbm.at[pl.ds(base, _IPW)], lab_v)
        lane = lax.iota(jnp.int32, 16)
        seg_row = lax.shift_right_logical(lane, 3)
        for seg in range(_IPW // 16):
            lv = lab_v[pl.ds(seg * 16, 16)]
            rows = wid * _RPW + seg * 2 + seg_row
            idx_v[pl.ds(seg * 16, 16)] = lv * _B + rows
        pltpu.async_copy(x_hbm.at[idx_v], val_v, sem).wait()
        g_acc = jnp.zeros((16,), jnp.float32)
        u_acc = jnp.zeros((16,), jnp.float32)
        for seg in range(_IPW // 16):
            k = idx_v[pl.ds(seg * 16, 16)]
            v = val_v[pl.ds(seg * 16, 16)]
            cnt = jnp.zeros((16,), jnp.float32)
            for t in range(16):
                s = k[t]
                cnt = cnt + jnp.where(k == s, 1.0, 0.0)
            inv = 1.0 / cnt
            g_acc = g_acc + v * inv
            u_acc = u_acc + inv
        gst_v[...] = g_acc
        ust_v[...] = u_acc
        pltpu.sync_copy(gst_v, g_hbm.at[wid])
        pltpu.sync_copy(ust_v, u_hbm.at[wid])

    return sc_kernel(labels_flat, out_flat)


_CHUNK = 5000
_NCHUNK = _V // _CHUNK
_NBUF = 5


def _tc_reduce_and_combine(xr, g_part, u_part):
    def body(x_hbm, g_ref, u_ref, o_ref, *scr):
        bufs, sems = scr[:_NBUF], scr[_NBUF:]

        def start(c):
            pltpu.make_async_copy(
                x_hbm.at[pl.ds(c * _CHUNK, _CHUNK)], bufs[c % _NBUF],
                sems[c % _NBUF]).start()

        for c in range(_NBUF - 1):
            start(c)
        acc = jnp.zeros((8, _B), jnp.float32)
        for c in range(_NCHUNK):
            pltpu.make_async_copy(
                x_hbm.at[pl.ds(c * _CHUNK, _CHUNK)], bufs[c % _NBUF],
                sems[c % _NBUF]).wait()
            nxt = c + _NBUF - 1
            if nxt < _NCHUNK:
                start(nxt)
            acc = acc + jnp.sum(
                bufs[c % _NBUF][...].reshape(_CHUNK // 8, 8, _B), axis=0)
        s_tot = jnp.sum(acc)
        g_tot = jnp.sum(g_ref[...])
        u_tot = jnp.sum(u_ref[...])
        o_ref[0, 0] = (u_tot * _C_UNIQ + _C_ENT
                       - _BASE * s_tot - (_HI - _BASE) * g_tot) / _B

    return pl.pallas_call(
        body,
        in_specs=[
            pl.BlockSpec(memory_space=pl.ANY),
            pl.BlockSpec(memory_space=pltpu.VMEM),
            pl.BlockSpec(memory_space=pltpu.VMEM),
        ],
        out_specs=pl.BlockSpec(memory_space=pltpu.SMEM),
        out_shape=jax.ShapeDtypeStruct((1, 1), jnp.float32),
        scratch_shapes=(
            [pltpu.VMEM((_CHUNK, _B), jnp.float32) for _ in range(_NBUF)]
            + [pltpu.SemaphoreType.DMA for _ in range(_NBUF)]
        ),
    )(xr, g_part, u_part)


def kernel(mode, output, output_m, batch_labels):
    del mode, output_m
    out_t = output.T
    out_flat = out_t.reshape(_V * _B)
    labels_flat = batch_labels.astype(jnp.int32).reshape(_B * _L)
    g_part, u_part = _sc_label_partials(labels_flat, out_flat)
    loss = _tc_reduce_and_combine(out_t, g_part, u_part)
    return loss[0, 0]


# speedup vs baseline: 1.0221x; 1.0221x over previous
import functools
import math

import jax
import jax.numpy as jnp
import numpy as np
from jax import lax
from jax.experimental import pallas as pl
from jax.experimental.pallas import tpu as pltpu
from jax.experimental.pallas import tpu_sc as plsc

_B = 128
_V = 100000
_L = 8
_SMOOTHING = 0.1
_BASE = float(np.float32(_SMOOTHING / (_V - _L)))
_HI = float(np.float32((1.0 - _SMOOTHING) / _L))
_C_UNIQ = _HI * math.log(_HI) - _BASE * math.log(_BASE)
_C_ENT = _B * _V * _BASE * math.log(_BASE)

_NC, _NS = 2, 16
_NW = _NC * _NS
_RPW = _B // _NW
_IPW = _RPW * _L

_RBLK = 25000
_NBLK = _V // _RBLK


def _sc_label_partials(labels_flat, out_flat):
    mesh = plsc.VectorSubcoreMesh(core_axis_name="c", subcore_axis_name="s")

    @functools.partial(
        pl.kernel,
        mesh=mesh,
        out_type=(
            jax.ShapeDtypeStruct((_NW, 16), jnp.float32),
            jax.ShapeDtypeStruct((_NW, 16), jnp.float32),
        ),
        scratch_types=[
            pltpu.VMEM((_IPW,), jnp.int32),
            pltpu.VMEM((_IPW,), jnp.int32),
            pltpu.VMEM((_IPW,), jnp.float32),
            pltpu.VMEM((16,), jnp.float32),
            pltpu.VMEM((16,), jnp.float32),
            pltpu.SemaphoreType.DMA,
        ],
    )
    def sc_kernel(labels_hbm, x_hbm, g_hbm, u_hbm,
                  lab_v, idx_v, val_v, gst_v, ust_v, sem):
        wid = lax.axis_index("s") * _NC + lax.axis_index("c")
        base = wid * _IPW
        pltpu.sync_copy(labels_hbm.at[pl.ds(base, _IPW)], lab_v)
        lane = lax.iota(jnp.int32, 16)
        seg_row = lax.shift_right_logical(lane, 3)
        for seg in range(_IPW // 16):
            lv = lab_v[pl.ds(seg * 16, 16)]
            rows = wid * _RPW + seg * 2 + seg_row
            idx_v[pl.ds(seg * 16, 16)] = lv * _B + rows
        pltpu.async_copy(x_hbm.at[idx_v], val_v, sem).wait()
        g_acc = jnp.zeros((16,), jnp.float32)
        u_acc = jnp.zeros((16,), jnp.float32)
        for seg in range(_IPW // 16):
            k = idx_v[pl.ds(seg * 16, 16)]
            v = val_v[pl.ds(seg * 16, 16)]
            cnt = jnp.zeros((16,), jnp.float32)
            for t in range(16):
                s = k[t]
                cnt = cnt + jnp.where(k == s, 1.0, 0.0)
            inv = 1.0 / cnt
            g_acc = g_acc + v * inv
            u_acc = u_acc + inv
        gst_v[...] = g_acc
        ust_v[...] = u_acc
        pltpu.sync_copy(gst_v, g_hbm.at[wid])
        pltpu.sync_copy(ust_v, u_hbm.at[wid])

    return sc_kernel(labels_flat, out_flat)


_CHUNK = 5000
_NCHUNK = _V // _CHUNK
_NBUF = 4


def _tc_reduce_and_combine(xr, g_part, u_part):
    def body(x_hbm, g_ref, u_ref, o_ref, *scr):
        bufs, sems = scr[:_NBUF], scr[_NBUF:]

        def start(c):
            pltpu.make_async_copy(
                x_hbm.at[pl.ds(c * _CHUNK, _CHUNK)], bufs[c % _NBUF],
                sems[c % _NBUF]).start()

        for c in range(_NBUF - 1):
            start(c)
        acc = jnp.zeros((8, _B), jnp.float32)
        for c in range(_NCHUNK):
            pltpu.make_async_copy(
                x_hbm.at[pl.ds(c * _CHUNK, _CHUNK)], bufs[c % _NBUF],
                sems[c % _NBUF]).wait()
            nxt = c + _NBUF - 1
            if nxt < _NCHUNK:
                start(nxt)
            acc = acc + jnp.sum(
                bufs[c % _NBUF][...].reshape(_CHUNK // 8, 8, _B), axis=0)
        s_tot = jnp.sum(acc)
        g_tot = jnp.sum(g_ref[...])
        u_tot = jnp.sum(u_ref[...])
        o_ref[0, 0] = (u_tot * _C_UNIQ + _C_ENT
                       - _BASE * s_tot - (_HI - _BASE) * g_tot) / _B

    return pl.pallas_call(
        body,
        in_specs=[
            pl.BlockSpec(memory_space=pl.ANY),
            pl.BlockSpec(memory_space=pltpu.VMEM),
            pl.BlockSpec(memory_space=pltpu.VMEM),
        ],
        out_specs=pl.BlockSpec(memory_space=pltpu.SMEM),
        out_shape=jax.ShapeDtypeStruct((1, 1), jnp.float32),
        scratch_shapes=(
            [pltpu.VMEM((_CHUNK, _B), jnp.float32) for _ in range(_NBUF)]
            + [pltpu.SemaphoreType.DMA for _ in range(_NBUF)]
        ),
    )(xr, g_part, u_part)


def kernel(mode, output, output_m, batch_labels):
    del mode, output_m
    out_t = output.T
    out_flat = out_t.reshape(_V * _B)
    labels_flat = batch_labels.astype(jnp.int32).reshape(_B * _L)
    g_part, u_part = _sc_label_partials(labels_flat, out_flat)
    loss = _tc_reduce_and_combine(out_t, g_part, u_part)
    return loss[0, 0]
